# baseline (device time: 18915 ns/iter reference)
import jax
import jax.numpy as jnp
from jax import lax
from jax.experimental import pallas as pl
from jax.experimental.pallas import tpu as pltpu

N_DEV = 16
N_GLOBAL = 16384
EPS = 1e-5
R, C = 16, 128
N_CHUNK = 8


def kernel(x, gamma):
    m, n_per = x.shape
    rows = m // N_CHUNK
    r_per = R // N_CHUNK

    def body(x_hbm, g_ref, out_hbm, x_vmem, o_vmem, comm_ref,
             in_sems, out_sems, send_sems, recv_sems):
        my = lax.axis_index("i")

        barrier_sem = pltpu.get_barrier_semaphore()
        for o in range(1, N_DEV):
            pl.semaphore_signal(
                barrier_sem,
                inc=1,
                device_id=(lax.rem(my + o, N_DEV),),
                device_id_type=pl.DeviceIdType.MESH,
            )

        in_copies = []
        for c in range(N_CHUNK):
            cp = pltpu.make_async_copy(
                x_hbm.at[pl.ds(c * rows, rows)],
                x_vmem.at[pl.ds(c * rows, rows)],
                in_sems.at[c],
            )
            cp.start()
            in_copies.append(cp)

        for c in range(N_CHUNK):
            in_copies[c].wait()
            xc = x_vmem[pl.ds(c * rows, rows), :].reshape(r_per, C, n_per)
            comm_ref[0, pl.ds(c * r_per, r_per), :] = jnp.sum(xc * xc, axis=2)

        pl.semaphore_wait(barrier_sem, N_DEV - 1)

        rdmas = []
        for o in range(1, N_DEV):
            rdma = pltpu.make_async_remote_copy(
                src_ref=comm_ref.at[0],
                dst_ref=comm_ref.at[o],
                send_sem=send_sems.at[o],
                recv_sem=recv_sems.at[o],
                device_id=(lax.rem(my + o, N_DEV),),
                device_id_type=pl.DeviceIdType.MESH,
            )
            rdma.start()
            rdmas.append(rdma)

        g = g_ref[...]
        for c in range(N_CHUNK):
            o_vmem[pl.ds(c * rows, rows), :] = (
                x_vmem[pl.ds(c * rows, rows), :] * g
            )

        for rdma in rdmas:
            rdma.wait_recv()

        total = jnp.sum(comm_ref[...], axis=0)
        inv = lax.rsqrt(total / N_GLOBAL + EPS)

        out_copies = []
        for c in range(N_CHUNK):
            sl = pl.ds(c * rows, rows)
            xg = o_vmem[sl, :].reshape(r_per, C, n_per)
            invc = inv[c * r_per : (c + 1) * r_per, :]
            o_vmem[sl, :] = (xg * invc[:, :, None]).reshape(rows, n_per)
            cp = pltpu.make_async_copy(
                o_vmem.at[sl], out_hbm.at[sl], out_sems.at[c]
            )
            cp.start()
            out_copies.append(cp)

        for cp in out_copies:
            cp.wait()
        for rdma in rdmas:
            rdma.wait_send()

    return pl.pallas_call(
        body,
        out_shape=jax.ShapeDtypeStruct((m, n_per), jnp.float32),
        in_specs=[
            pl.BlockSpec(memory_space=pl.ANY),
            pl.BlockSpec(memory_space=pltpu.VMEM),
        ],
        out_specs=pl.BlockSpec(memory_space=pl.ANY),
        scratch_shapes=[
            pltpu.VMEM((m, n_per), jnp.float32),
            pltpu.VMEM((m, n_per), jnp.float32),
            pltpu.VMEM((N_DEV, R, C), jnp.float32),
            pltpu.SemaphoreType.DMA((N_CHUNK,)),
            pltpu.SemaphoreType.DMA((N_CHUNK,)),
            pltpu.SemaphoreType.DMA((N_DEV,)),
            pltpu.SemaphoreType.DMA((N_DEV,)),
        ],
        compiler_params=pltpu.CompilerParams(collective_id=0),
    )(x, gamma.reshape(1, n_per))
